# Initial kernel scaffold; baseline (speedup 1.0000x reference)
import jax
import jax.numpy as jnp
from jax.experimental import pallas as pl


def kernel(x, atomic_numbers, shift):
    raise NotImplementedError("write your pallas kernel here")


# speedup vs baseline: 286.6083x; 286.6083x over previous
import functools

import jax
import jax.numpy as jnp
from jax import lax
from jax.experimental import pallas as pl
from jax.experimental.pallas import tpu as pltpu
from jax.experimental.pallas import tpu_sc as plsc

_N = 1048576
_N_SPECIES = 119
_TAB = 128
_NC, _NS, _L = 2, 16, 16
_NW = _NC * _NS
_PER_W = _N // _NW


def _build():
    mesh = plsc.VectorSubcoreMesh(core_axis_name="c", subcore_axis_name="s")

    @functools.partial(
        pl.kernel,
        mesh=mesh,
        compiler_params=pltpu.CompilerParams(needs_layout_passes=False),
        out_type=jax.ShapeDtypeStruct((_N,), jnp.float32),
        scratch_types=[
            pltpu.VMEM((_TAB,), jnp.float32),
            pltpu.VMEM((_PER_W,), jnp.int32),
            pltpu.VMEM((_PER_W,), jnp.float32),
        ],
    )
    def k(x_hbm, idx_hbm, shift_hbm, out_hbm, table_v, idx_v, x_v):
        wid = lax.axis_index("s") * _NC + lax.axis_index("c")
        base = wid * _PER_W
        pltpu.sync_copy(shift_hbm, table_v)
        pltpu.sync_copy(idx_hbm.at[pl.ds(base, _PER_W)], idx_v)
        pltpu.sync_copy(x_hbm.at[pl.ds(base, _PER_W)], x_v)

        def body(i, carry):
            sl = pl.ds(i * _L, _L)
            iv = idx_v[sl]
            sv = plsc.load_gather(table_v, [iv])
            x_v[sl] = x_v[sl] - sv
            return carry

        lax.fori_loop(0, _PER_W // _L, body, 0)
        pltpu.sync_copy(x_v, out_hbm.at[pl.ds(base, _PER_W)])

    return k


_sc_kernel = _build()


def kernel(x, atomic_numbers, shift):
    idx = atomic_numbers.astype(jnp.int32)
    table = jnp.pad(shift.reshape(-1), (0, _TAB - _N_SPECIES))
    return _sc_kernel(x, idx, table)
